# baseline (speedup 1.0000x reference)
import functools

import jax
import jax.numpy as jnp
from jax import lax
from jax.experimental import pallas as pl
from jax.experimental.pallas import tpu as pltpu
from jax.experimental.pallas import tpu_sc as plsc

NC = 2
NS = 16
NW = NC * NS
CH = 128
NPAD = 512


def _tile_node_range(s, n, body):
    base = (n // NS) // 8 * 8
    last = n - base * (NS - 1)

    @pl.when(s < NS - 1)
    def _():
        body(pl.multiple_of(s * base, 8), base)

    @pl.when(s == NS - 1)
    def _():
        body((NS - 1) * base, last)


def _sc_degrees(idx_flat, ones1, zeros1, n_pad):
    ept2 = idx_flat.shape[0] // NW
    cpt2 = ept2 // CH
    n2 = 2 * n_pad

    mesh = plsc.VectorSubcoreMesh(core_axis_name="c", subcore_axis_name="s")
    K = 4
    assert cpt2 % (2 * K) == 0
    nj = cpt2 // (2 * K)

    ix_scratch = {f"ix{b}{m}": pltpu.VMEM((CH,), jnp.int32)
                  for b in range(2) for m in range(K)}
    sem_scratch = {f"sx{b}{m}": pltpu.SemaphoreType.DMA
                   for b in range(2) for m in range(K)}
    sem_scratch.update({f"ss{b}{m}": pltpu.SemaphoreType.DMA
                        for b in range(2) for m in range(K)})

    @functools.partial(
        pl.kernel,
        mesh=mesh,
        out_type=jax.ShapeDtypeStruct((NC * n2,), jnp.float32),
        scratch_types=dict(
            ones_v=pltpu.VMEM((CH,), jnp.float32),
            zb=pltpu.VMEM((n2 - (NS - 1) * ((n2 // NS) // 8 * 8),),
                          jnp.float32),
            degs_sh=pltpu.VMEM_SHARED((n2,), jnp.float32),
            **ix_scratch,
            **sem_scratch,
        ),
    )
    def deg_kernel(idx_hbm, ones_hbm, z1_hbm, degs_hbm,
                   ones_v, zb, degs_sh, **refs):
        c = lax.axis_index("c")
        s = lax.axis_index("s")
        wid = c * NS + s
        ebase = wid * ept2
        ix = [[refs[f"ix{b}{m}"] for m in range(K)] for b in range(2)]
        sx = [[refs[f"sx{b}{m}"] for m in range(K)] for b in range(2)]
        ss = [[refs[f"ss{b}{m}"] for m in range(K)] for b in range(2)]
        pltpu.sync_copy(ones_hbm, ones_v)

        def zero_rows(r0, npt):
            pltpu.sync_copy(z1_hbm.at[pl.ds(r0, npt)], zb.at[pl.ds(0, npt)])
            pltpu.sync_copy(zb.at[pl.ds(0, npt)], degs_sh.at[pl.ds(r0, npt)])

        _tile_node_range(s, n2, zero_rows)
        plsc.subcore_barrier()

        def load_ix(chunk, b, m):
            pltpu.async_copy(idx_hbm.at[pl.ds(ebase + chunk * CH, CH)],
                             ix[b][m], sx[b][m])

        def wait_ix(chunk, b, m):
            pltpu.make_async_copy(idx_hbm.at[pl.ds(ebase + chunk * CH, CH)],
                                  ix[b][m], sx[b][m]).wait()

        def fire_scat(b, m):
            pltpu.async_copy(ones_v, degs_sh.at[ix[b][m]], ss[b][m],
                             add=True)

        def drain_scat(b, m):
            pltpu.make_async_copy(ones_v, degs_sh.at[ix[b][m]],
                                  ss[b][m]).wait()

        for m in range(K):
            load_ix(m, 0, m)

        def step(j, carry):
            c0 = 2 * K * j
            for m in range(K):
                wait_ix(c0 + m, 0, m)
                fire_scat(0, m)
            for m in range(K):
                load_ix(c0 + K + m, 1, m)
            for m in range(K):
                drain_scat(0, m)

                @pl.when(j < nj - 1)
                def _():
                    load_ix(c0 + 2 * K + m, 0, m)

            for m in range(K):
                wait_ix(c0 + K + m, 1, m)
                fire_scat(1, m)
            for m in range(K):
                drain_scat(1, m)
            return carry

        lax.fori_loop(0, nj, step, 0)
        plsc.subcore_barrier()

        def write_rows(r0, npt):
            pltpu.sync_copy(degs_sh.at[pl.ds(r0, npt)], zb.at[pl.ds(0, npt)])
            pltpu.sync_copy(zb.at[pl.ds(0, npt)],
                            degs_hbm.at[pl.ds(c * n2 + r0, npt)])

        _tile_node_range(s, n2, write_rows)

    return deg_kernel(idx_flat, ones1, zeros1)


def _sc_segsum(h_pad, src2, dst_flat, zeros, n_pad, d):
    n_chunks, ch = src2.shape
    cpt = n_chunks // NW

    mesh = plsc.VectorSubcoreMesh(core_axis_name="c", subcore_axis_name="s")

    @functools.partial(
        pl.kernel,
        mesh=mesh,
        out_type=jax.ShapeDtypeStruct((NC * n_pad, d), jnp.float32),
        scratch_types=dict(
            isrc=pltpu.VMEM((cpt, ch), jnp.int32),
            x0=pltpu.VMEM((ch,), jnp.int32),
            x1=pltpu.VMEM((ch,), jnp.int32),
            x2=pltpu.VMEM((ch,), jnp.int32),
            x3=pltpu.VMEM((ch,), jnp.int32),
            rows_a=pltpu.VMEM((ch, d), jnp.float32),
            rows_b=pltpu.VMEM((ch, d), jnp.float32),
            acc_sh=pltpu.VMEM_SHARED((n_pad, d), jnp.float32),
            sga=pltpu.SemaphoreType.DMA,
            sgb=pltpu.SemaphoreType.DMA,
            ssa=pltpu.SemaphoreType.DMA,
            ssb=pltpu.SemaphoreType.DMA,
            sx0=pltpu.SemaphoreType.DMA,
            sx1=pltpu.SemaphoreType.DMA,
            sx2=pltpu.SemaphoreType.DMA,
            sx3=pltpu.SemaphoreType.DMA,
        ),
    )
    def seg_kernel(h_hbm, src_hbm, dstf_hbm, z_hbm, out_hbm,
                   isrc, x0, x1, x2, x3, rows_a, rows_b, acc_sh,
                   sga, sgb, ssa, ssb, sx0, sx1, sx2, sx3):
        c = lax.axis_index("c")
        s = lax.axis_index("s")
        wid = c * NS + s
        row0 = wid * cpt
        ebase = wid * cpt * ch
        pltpu.sync_copy(src_hbm.at[pl.ds(row0, cpt)], isrc)
        _tile_node_range(s, n_pad, lambda r0, npt: pltpu.sync_copy(
            z_hbm.at[pl.ds(r0, npt)], acc_sh.at[pl.ds(r0, npt)]))
        plsc.subcore_barrier()

        xs = [(x0, sx0), (x1, sx1), (x2, sx2), (x3, sx3)]
        rws = [(rows_a, sga, ssa), (rows_b, sgb, ssb)]

        def load_ix(i, m):
            pltpu.async_copy(dstf_hbm.at[pl.ds(ebase + i * ch, ch)],
                             xs[m][0], xs[m][1])

        def wait_ix(i, m):
            pltpu.make_async_copy(dstf_hbm.at[pl.ds(ebase + i * ch, ch)],
                                  xs[m][0], xs[m][1]).wait()

        def fire_g(i, p):
            pltpu.async_copy(h_hbm.at[isrc.at[i]], rws[p][0], rws[p][1])

        def wait_g(i, p):
            pltpu.make_async_copy(h_hbm.at[isrc.at[i]],
                                  rws[p][0], rws[p][1]).wait()

        def fire_s(m, p):
            pltpu.async_copy(rws[p][0], acc_sh.at[xs[m][0]], rws[p][2],
                             add=True)

        def drain_s(m, p):
            pltpu.make_async_copy(rws[p][0], acc_sh.at[xs[m][0]],
                                  rws[p][2]).wait()

        for m in range(4):
            load_ix(m, m)
        fire_g(0, 0)

        def step(j, carry):
            c0 = 4 * j
            wait_ix(c0, 0)
            wait_g(c0, 0)
            fire_s(0, 0)

            @pl.when(j > 0)
            def _():
                drain_s(3, 1)
                load_ix(c0 + 3, 3)

            fire_g(c0 + 1, 1)
            wait_ix(c0 + 1, 1)
            wait_g(c0 + 1, 1)
            fire_s(1, 1)
            drain_s(0, 0)

            @pl.when(c0 + 4 < cpt)
            def _():
                load_ix(c0 + 4, 0)

            fire_g(c0 + 2, 0)
            wait_ix(c0 + 2, 2)
            wait_g(c0 + 2, 0)
            fire_s(2, 0)
            drain_s(1, 1)

            @pl.when(c0 + 5 < cpt)
            def _():
                load_ix(c0 + 5, 1)

            fire_g(c0 + 3, 1)
            wait_ix(c0 + 3, 3)
            wait_g(c0 + 3, 1)
            fire_s(3, 1)
            drain_s(2, 0)

            @pl.when(c0 + 6 < cpt)
            def _():
                load_ix(c0 + 6, 2)

            @pl.when(j < cpt // 4 - 1)
            def _():
                fire_g(c0 + 4, 0)

            return carry

        lax.fori_loop(0, cpt // 4, step, 0)
        drain_s(3, 1)
        plsc.subcore_barrier()
        _tile_node_range(s, n_pad, lambda r0, npt: pltpu.sync_copy(
            acc_sh.at[pl.ds(r0, npt)],
            out_hbm.at[pl.ds(c * n_pad + r0, npt)]))

    return seg_kernel(h_pad, src2, dst_flat, zeros)


def _norms(degblk):
    ns = lax.rsqrt(jnp.clip(degblk[0] + degblk[1], 1.0, None))
    nd = lax.rsqrt(jnp.clip(degblk[2] + degblk[3], 1.0, None))
    return ns, nd


def _tc_matmul(x, w, n, d, bn):
    def body(x_ref, w_ref, o_ref):
        o_ref[...] = jnp.dot(x_ref[...], w_ref[...],
                             preferred_element_type=jnp.float32)

    return pl.pallas_call(
        body,
        grid=(n // bn,),
        in_specs=[
            pl.BlockSpec((bn, d), lambda i: (i, 0)),
            pl.BlockSpec((d, d), lambda i: (0, 0)),
        ],
        out_specs=pl.BlockSpec((bn, d), lambda i: (i, 0)),
        out_shape=jax.ShapeDtypeStruct((n, d), jnp.float32),
    )(x, w)


def _tc_prescale(y, degs_tc, n, d, bn, n_out):
    def body(deg_ref, x_ref, o_ref):
        ns, _ = _norms(deg_ref[0])
        o_ref[...] = x_ref[...] * ns[:, None]

    return pl.pallas_call(
        body,
        grid=(n // bn,),
        in_specs=[
            pl.BlockSpec((1, 4, bn), lambda i: (i, 0, 0)),
            pl.BlockSpec((bn, d), lambda i: (i, 0)),
        ],
        out_specs=pl.BlockSpec((bn, d), lambda i: (i, 0)),
        out_shape=jax.ShapeDtypeStruct((n_out, d), jnp.float32),
    )(degs_tc, y)


def _tc_mid(part, degs_tc, w2, b1, n, d, bn, n_out):
    def body(p_ref, deg_ref, w_ref, b_ref, o_ref):
        ns, nd = _norms(deg_ref[0])
        h2 = jnp.maximum((p_ref[0] + p_ref[1]) * nd[:, None] + b_ref[...],
                         0.0)
        o_ref[...] = jnp.dot(
            h2, w_ref[...], preferred_element_type=jnp.float32) * ns[:, None]

    return pl.pallas_call(
        body,
        grid=(n // bn,),
        in_specs=[
            pl.BlockSpec((NC, bn, d), lambda i: (0, i, 0)),
            pl.BlockSpec((1, 4, bn), lambda i: (i, 0, 0)),
            pl.BlockSpec((d, d), lambda i: (0, 0)),
            pl.BlockSpec((1, d), lambda i: (0, 0)),
        ],
        out_specs=pl.BlockSpec((bn, d), lambda i: (i, 0)),
        out_shape=jax.ShapeDtypeStruct((n_out, d), jnp.float32),
    )(part, degs_tc, w2, b1)


def _tc_final(part, degs_tc, b, n, d, bn):
    def body(p_ref, deg_ref, b_ref, o_ref):
        _, nd = _norms(deg_ref[0])
        o_ref[...] = (p_ref[0] + p_ref[1]) * nd[:, None] + b_ref[...]

    return pl.pallas_call(
        body,
        grid=(n // bn,),
        in_specs=[
            pl.BlockSpec((NC, bn, d), lambda i: (0, i, 0)),
            pl.BlockSpec((1, 4, bn), lambda i: (i, 0, 0)),
            pl.BlockSpec((1, d), lambda i: (0, 0)),
        ],
        out_specs=pl.BlockSpec((bn, d), lambda i: (i, 0)),
        out_shape=jax.ShapeDtypeStruct((n, d), jnp.float32),
    )(part, degs_tc, b)


def _pad_edges(idx, e_pad, n):
    pad = e_pad - idx.shape[0]
    pidx = (n + (jnp.arange(pad, dtype=jnp.int32) % NPAD)).astype(idx.dtype)
    return jnp.concatenate([idx, pidx])


def kernel(feat, edge_index, W1, b1, W2, b2):
    n, d = feat.shape
    e = edge_index.shape[1]
    n_pad = n + NPAD

    cpt = -(-e // (NW * CH))
    cpt += cpt % 2
    e_pad = NW * cpt * CH

    src_flat = _pad_edges(edge_index[0], e_pad, n)
    dst_flat = _pad_edges(edge_index[1], e_pad, n)
    src2 = src_flat.reshape(e_pad // CH, CH)
    dst2 = dst_flat.reshape(e_pad // CH, CH)
    idx_flat = jnp.stack([src2, dst2 + n_pad], axis=1).reshape(-1)
    zeros = jnp.zeros((n_pad, d), jnp.float32)
    zeros1 = jnp.zeros((2 * n_pad,), jnp.float32)
    ones1 = jnp.ones((CH,), jnp.float32)
    b1r = b1.reshape(1, d)
    b2r = b2.reshape(1, d)

    bn = 2000
    y1 = _tc_matmul(feat, W1, n, d, bn)
    degs = _sc_degrees(idx_flat, ones1, zeros1, n_pad).reshape(NC, 2, n_pad)
    degs_tc = (degs[:, :, :n].transpose(1, 0, 2).reshape(4, n // bn, bn)
               .transpose(1, 0, 2))

    h1p = _tc_prescale(y1, degs_tc, n, d, bn, n_pad)
    p1 = _sc_segsum(h1p, src2, dst_flat, zeros, n_pad, d)
    p1 = p1.reshape(NC, n_pad, d)
    q2p = _tc_mid(p1, degs_tc, W2, b1r, n, d, bn, n_pad)
    p2 = _sc_segsum(q2p, src2, dst_flat, zeros, n_pad, d)
    p2 = p2.reshape(NC, n_pad, d)
    out = _tc_final(p2, degs_tc, b2r, n, d, bn)
    return out


# speedup vs baseline: 1.1525x; 1.1525x over previous
import functools

import jax
import jax.numpy as jnp
from jax import lax
from jax.experimental import pallas as pl
from jax.experimental.pallas import tpu as pltpu
from jax.experimental.pallas import tpu_sc as plsc

NC = 2
NS = 16
NW = NC * NS
CH = 128
NPAD = 512


def _tile_node_range(s, n, body):
    base = (n // NS) // 8 * 8
    last = n - base * (NS - 1)

    @pl.when(s < NS - 1)
    def _():
        body(pl.multiple_of(s * base, 8), base)

    @pl.when(s == NS - 1)
    def _():
        body((NS - 1) * base, last)


def _sc_degrees(idx_flat, ones1, zeros1, n_pad):
    ept2 = idx_flat.shape[0] // NW
    cpt2 = ept2 // CH
    n2 = 2 * n_pad

    mesh = plsc.VectorSubcoreMesh(core_axis_name="c", subcore_axis_name="s")
    K = 4
    assert cpt2 % (2 * K) == 0
    nj = cpt2 // (2 * K)

    ix_scratch = {f"ix{b}{m}": pltpu.VMEM((CH,), jnp.int32)
                  for b in range(2) for m in range(K)}
    sem_scratch = {f"sx{b}{m}": pltpu.SemaphoreType.DMA
                   for b in range(2) for m in range(K)}
    sem_scratch.update({f"ss{b}{m}": pltpu.SemaphoreType.DMA
                        for b in range(2) for m in range(K)})

    @functools.partial(
        pl.kernel,
        mesh=mesh,
        out_type=jax.ShapeDtypeStruct((NC * n2,), jnp.float32),
        scratch_types=dict(
            ones_v=pltpu.VMEM((CH,), jnp.float32),
            zb=pltpu.VMEM((n2 - (NS - 1) * ((n2 // NS) // 8 * 8),),
                          jnp.float32),
            degs_sh=pltpu.VMEM_SHARED((n2,), jnp.float32),
            **ix_scratch,
            **sem_scratch,
        ),
    )
    def deg_kernel(idx_hbm, ones_hbm, z1_hbm, degs_hbm,
                   ones_v, zb, degs_sh, **refs):
        c = lax.axis_index("c")
        s = lax.axis_index("s")
        wid = c * NS + s
        ebase = wid * ept2
        ix = [[refs[f"ix{b}{m}"] for m in range(K)] for b in range(2)]
        sx = [[refs[f"sx{b}{m}"] for m in range(K)] for b in range(2)]
        ss = [[refs[f"ss{b}{m}"] for m in range(K)] for b in range(2)]
        pltpu.sync_copy(ones_hbm, ones_v)

        def zero_rows(r0, npt):
            pltpu.sync_copy(z1_hbm.at[pl.ds(r0, npt)], zb.at[pl.ds(0, npt)])
            pltpu.sync_copy(zb.at[pl.ds(0, npt)], degs_sh.at[pl.ds(r0, npt)])

        _tile_node_range(s, n2, zero_rows)
        plsc.subcore_barrier()

        def load_ix(chunk, b, m):
            pltpu.async_copy(idx_hbm.at[pl.ds(ebase + chunk * CH, CH)],
                             ix[b][m], sx[b][m])

        def wait_ix(chunk, b, m):
            pltpu.make_async_copy(idx_hbm.at[pl.ds(ebase + chunk * CH, CH)],
                                  ix[b][m], sx[b][m]).wait()

        def fire_scat(b, m):
            pltpu.async_copy(ones_v, degs_sh.at[ix[b][m]], ss[b][m],
                             add=True)

        def drain_scat(b, m):
            pltpu.make_async_copy(ones_v, degs_sh.at[ix[b][m]],
                                  ss[b][m]).wait()

        for m in range(K):
            load_ix(m, 0, m)

        def step(j, carry):
            c0 = 2 * K * j
            for m in range(K):
                wait_ix(c0 + m, 0, m)
                fire_scat(0, m)
            for m in range(K):
                load_ix(c0 + K + m, 1, m)
            for m in range(K):
                drain_scat(0, m)

                @pl.when(j < nj - 1)
                def _():
                    load_ix(c0 + 2 * K + m, 0, m)

            for m in range(K):
                wait_ix(c0 + K + m, 1, m)
                fire_scat(1, m)
            for m in range(K):
                drain_scat(1, m)
            return carry

        lax.fori_loop(0, nj, step, 0)
        plsc.subcore_barrier()

        def write_rows(r0, npt):
            pltpu.sync_copy(degs_sh.at[pl.ds(r0, npt)], zb.at[pl.ds(0, npt)])
            pltpu.sync_copy(zb.at[pl.ds(0, npt)],
                            degs_hbm.at[pl.ds(c * n2 + r0, npt)])

        _tile_node_range(s, n2, write_rows)

    return deg_kernel(idx_flat, ones1, zeros1)


def _sc_segsum(h_pad, src2, dst_flat, zeros, n_pad, d):
    n_chunks, ch = src2.shape
    cpt = n_chunks // NW

    mesh = plsc.VectorSubcoreMesh(core_axis_name="c", subcore_axis_name="s")

    @functools.partial(
        pl.kernel,
        mesh=mesh,
        out_type=jax.ShapeDtypeStruct((NC * n_pad, d), jnp.float32),
        scratch_types=dict(
            isrc=pltpu.VMEM((cpt, ch), jnp.int32),
            ixa=pltpu.VMEM((ch,), jnp.int32),
            ixb=pltpu.VMEM((ch,), jnp.int32),
            rows_a=pltpu.VMEM((ch, d), jnp.float32),
            rows_b=pltpu.VMEM((ch, d), jnp.float32),
            acc_sh=pltpu.VMEM_SHARED((n_pad, d), jnp.float32),
            sem_a=pltpu.SemaphoreType.DMA,
            sem_b=pltpu.SemaphoreType.DMA,
            sxa=pltpu.SemaphoreType.DMA,
            sxb=pltpu.SemaphoreType.DMA,
        ),
    )
    def seg_kernel(h_hbm, src_hbm, dstf_hbm, z_hbm, out_hbm,
                   isrc, ixa, ixb, rows_a, rows_b, acc_sh,
                   sem_a, sem_b, sxa, sxb):
        c = lax.axis_index("c")
        s = lax.axis_index("s")
        wid = c * NS + s
        row0 = wid * cpt
        ebase = wid * cpt * ch
        pltpu.sync_copy(src_hbm.at[pl.ds(row0, cpt)], isrc)

        def load_ix(i, ix, sx):
            pltpu.async_copy(dstf_hbm.at[pl.ds(ebase + i * ch, ch)], ix, sx)

        def wait_ix(i, ix, sx):
            pltpu.make_async_copy(dstf_hbm.at[pl.ds(ebase + i * ch, ch)],
                                  ix, sx).wait()

        load_ix(0, ixa, sxa)
        load_ix(1, ixb, sxb)
        pltpu.async_copy(h_hbm.at[isrc.at[0]], rows_a, sem_a)
        _tile_node_range(s, n_pad, lambda r0, npt: pltpu.sync_copy(
            z_hbm.at[pl.ds(r0, npt)], acc_sh.at[pl.ds(r0, npt)]))
        plsc.subcore_barrier()

        def do_chunk(i, rows, sem, ix, sx):
            wait_ix(i, ix, sx)
            pltpu.make_async_copy(h_hbm.at[isrc.at[i]], rows, sem).wait()
            pltpu.sync_copy(rows, acc_sh.at[ix], add=True)

            @pl.when(i + 2 < cpt)
            def _():
                load_ix(i + 2, ix, sx)

        def step(j, carry):
            i0 = 2 * j
            pltpu.async_copy(h_hbm.at[isrc.at[i0 + 1]], rows_b, sem_b)
            do_chunk(i0, rows_a, sem_a, ixa, sxa)

            @pl.when(j < cpt // 2 - 1)
            def _():
                pltpu.async_copy(h_hbm.at[isrc.at[i0 + 2]], rows_a, sem_a)

            do_chunk(i0 + 1, rows_b, sem_b, ixb, sxb)
            return carry

        lax.fori_loop(0, cpt // 2, step, 0)
        plsc.subcore_barrier()
        _tile_node_range(s, n_pad, lambda r0, npt: pltpu.sync_copy(
            acc_sh.at[pl.ds(r0, npt)],
            out_hbm.at[pl.ds(c * n_pad + r0, npt)]))

    return seg_kernel(h_pad, src2, dst_flat, zeros)


def _norms(degblk):
    ns = lax.rsqrt(jnp.clip(degblk[0] + degblk[1], 1.0, None))
    nd = lax.rsqrt(jnp.clip(degblk[2] + degblk[3], 1.0, None))
    return ns, nd


def _tc_matmul(x, w, n, d, bn):
    def body(x_ref, w_ref, o_ref):
        o_ref[...] = jnp.dot(x_ref[...], w_ref[...],
                             preferred_element_type=jnp.float32)

    return pl.pallas_call(
        body,
        grid=(n // bn,),
        in_specs=[
            pl.BlockSpec((bn, d), lambda i: (i, 0)),
            pl.BlockSpec((d, d), lambda i: (0, 0)),
        ],
        out_specs=pl.BlockSpec((bn, d), lambda i: (i, 0)),
        out_shape=jax.ShapeDtypeStruct((n, d), jnp.float32),
    )(x, w)


def _tc_prescale(y, degs_tc, n, d, bn, n_out):
    def body(deg_ref, x_ref, o_ref):
        ns, _ = _norms(deg_ref[0])
        o_ref[...] = x_ref[...] * ns[:, None]

    return pl.pallas_call(
        body,
        grid=(n // bn,),
        in_specs=[
            pl.BlockSpec((1, 4, bn), lambda i: (i, 0, 0)),
            pl.BlockSpec((bn, d), lambda i: (i, 0)),
        ],
        out_specs=pl.BlockSpec((bn, d), lambda i: (i, 0)),
        out_shape=jax.ShapeDtypeStruct((n_out, d), jnp.float32),
    )(degs_tc, y)


def _tc_mid(part, degs_tc, w2, b1, n, d, bn, n_out):
    def body(p_ref, deg_ref, w_ref, b_ref, o_ref):
        ns, nd = _norms(deg_ref[0])
        h2 = jnp.maximum((p_ref[0] + p_ref[1]) * nd[:, None] + b_ref[...],
                         0.0)
        o_ref[...] = jnp.dot(
            h2, w_ref[...], preferred_element_type=jnp.float32) * ns[:, None]

    return pl.pallas_call(
        body,
        grid=(n // bn,),
        in_specs=[
            pl.BlockSpec((NC, bn, d), lambda i: (0, i, 0)),
            pl.BlockSpec((1, 4, bn), lambda i: (i, 0, 0)),
            pl.BlockSpec((d, d), lambda i: (0, 0)),
            pl.BlockSpec((1, d), lambda i: (0, 0)),
        ],
        out_specs=pl.BlockSpec((bn, d), lambda i: (i, 0)),
        out_shape=jax.ShapeDtypeStruct((n_out, d), jnp.float32),
    )(part, degs_tc, w2, b1)


def _tc_final(part, degs_tc, b, n, d, bn):
    def body(p_ref, deg_ref, b_ref, o_ref):
        _, nd = _norms(deg_ref[0])
        o_ref[...] = (p_ref[0] + p_ref[1]) * nd[:, None] + b_ref[...]

    return pl.pallas_call(
        body,
        grid=(n // bn,),
        in_specs=[
            pl.BlockSpec((NC, bn, d), lambda i: (0, i, 0)),
            pl.BlockSpec((1, 4, bn), lambda i: (i, 0, 0)),
            pl.BlockSpec((1, d), lambda i: (0, 0)),
        ],
        out_specs=pl.BlockSpec((bn, d), lambda i: (i, 0)),
        out_shape=jax.ShapeDtypeStruct((n, d), jnp.float32),
    )(part, degs_tc, b)


def _pad_edges(idx, e_pad, n):
    pad = e_pad - idx.shape[0]
    pidx = (n + (jnp.arange(pad, dtype=jnp.int32) % NPAD)).astype(idx.dtype)
    return jnp.concatenate([idx, pidx])


def kernel(feat, edge_index, W1, b1, W2, b2):
    n, d = feat.shape
    e = edge_index.shape[1]
    n_pad = n + NPAD

    cpt = -(-e // (NW * CH))
    cpt += cpt % 2
    e_pad = NW * cpt * CH

    src_flat = _pad_edges(edge_index[0], e_pad, n)
    dst_flat = _pad_edges(edge_index[1], e_pad, n)
    src2 = src_flat.reshape(e_pad // CH, CH)
    dst2 = dst_flat.reshape(e_pad // CH, CH)
    idx_flat = jnp.stack([src2, dst2 + n_pad], axis=1).reshape(-1)
    zeros = jnp.zeros((n_pad, d), jnp.float32)
    zeros1 = jnp.zeros((2 * n_pad,), jnp.float32)
    ones1 = jnp.ones((CH,), jnp.float32)
    b1r = b1.reshape(1, d)
    b2r = b2.reshape(1, d)

    bn = 2000
    y1 = _tc_matmul(feat, W1, n, d, bn)
    degs = _sc_degrees(idx_flat, ones1, zeros1, n_pad).reshape(NC, 2, n_pad)
    degs_tc = (degs[:, :, :n].transpose(1, 0, 2).reshape(4, n // bn, bn)
               .transpose(1, 0, 2))

    h1p = _tc_prescale(y1, degs_tc, n, d, bn, n_pad)
    p1 = _sc_segsum(h1p, src2, dst_flat, zeros, n_pad, d)
    p1 = p1.reshape(NC, n_pad, d)
    q2p = _tc_mid(p1, degs_tc, W2, b1r, n, d, bn, n_pad)
    p2 = _sc_segsum(q2p, src2, dst_flat, zeros, n_pad, d)
    p2 = p2.reshape(NC, n_pad, d)
    out = _tc_final(p2, degs_tc, b2r, n, d, bn)
    return out


# speedup vs baseline: 1.1685x; 1.0139x over previous
import functools

import jax
import jax.numpy as jnp
from jax import lax
from jax.experimental import pallas as pl
from jax.experimental.pallas import tpu as pltpu
from jax.experimental.pallas import tpu_sc as plsc

NC = 2
NS = 16
NW = NC * NS
CH = 128
NPAD = 512


def _tile_node_range(s, n, body):
    base = (n // NS) // 8 * 8
    last = n - base * (NS - 1)

    @pl.when(s < NS - 1)
    def _():
        body(pl.multiple_of(s * base, 8), base)

    @pl.when(s == NS - 1)
    def _():
        body((NS - 1) * base, last)


def _sc_degrees(idx_flat, ones1, zeros1, n_pad):
    ept2 = idx_flat.shape[0] // NW
    cpt2 = ept2 // CH
    n2 = 2 * n_pad

    mesh = plsc.VectorSubcoreMesh(core_axis_name="c", subcore_axis_name="s")
    K = 4
    assert cpt2 % (2 * K) == 0
    nj = cpt2 // (2 * K)

    ix_scratch = {f"ix{b}{m}": pltpu.VMEM((CH,), jnp.int32)
                  for b in range(2) for m in range(K)}
    sem_scratch = {f"sx{b}{m}": pltpu.SemaphoreType.DMA
                   for b in range(2) for m in range(K)}
    sem_scratch.update({f"ss{b}{m}": pltpu.SemaphoreType.DMA
                        for b in range(2) for m in range(K)})

    @functools.partial(
        pl.kernel,
        mesh=mesh,
        out_type=jax.ShapeDtypeStruct((NC * n2,), jnp.float32),
        scratch_types=dict(
            ones_v=pltpu.VMEM((CH,), jnp.float32),
            zb=pltpu.VMEM((n2 - (NS - 1) * ((n2 // NS) // 8 * 8),),
                          jnp.float32),
            degs_sh=pltpu.VMEM_SHARED((n2,), jnp.float32),
            **ix_scratch,
            **sem_scratch,
        ),
    )
    def deg_kernel(idx_hbm, ones_hbm, z1_hbm, degs_hbm,
                   ones_v, zb, degs_sh, **refs):
        c = lax.axis_index("c")
        s = lax.axis_index("s")
        wid = c * NS + s
        ebase = wid * ept2
        ix = [[refs[f"ix{b}{m}"] for m in range(K)] for b in range(2)]
        sx = [[refs[f"sx{b}{m}"] for m in range(K)] for b in range(2)]
        ss = [[refs[f"ss{b}{m}"] for m in range(K)] for b in range(2)]
        pltpu.sync_copy(ones_hbm, ones_v)

        def zero_rows(r0, npt):
            pltpu.sync_copy(z1_hbm.at[pl.ds(r0, npt)], zb.at[pl.ds(0, npt)])
            pltpu.sync_copy(zb.at[pl.ds(0, npt)], degs_sh.at[pl.ds(r0, npt)])

        _tile_node_range(s, n2, zero_rows)
        plsc.subcore_barrier()

        def load_ix(chunk, b, m):
            pltpu.async_copy(idx_hbm.at[pl.ds(ebase + chunk * CH, CH)],
                             ix[b][m], sx[b][m])

        def wait_ix(chunk, b, m):
            pltpu.make_async_copy(idx_hbm.at[pl.ds(ebase + chunk * CH, CH)],
                                  ix[b][m], sx[b][m]).wait()

        def fire_scat(b, m):
            pltpu.async_copy(ones_v, degs_sh.at[ix[b][m]], ss[b][m],
                             add=True)

        def drain_scat(b, m):
            pltpu.make_async_copy(ones_v, degs_sh.at[ix[b][m]],
                                  ss[b][m]).wait()

        for m in range(K):
            load_ix(m, 0, m)

        def step(j, carry):
            c0 = 2 * K * j
            for m in range(K):
                wait_ix(c0 + m, 0, m)
                fire_scat(0, m)
            for m in range(K):
                load_ix(c0 + K + m, 1, m)
            for m in range(K):
                drain_scat(0, m)

                @pl.when(j < nj - 1)
                def _():
                    load_ix(c0 + 2 * K + m, 0, m)

            for m in range(K):
                wait_ix(c0 + K + m, 1, m)
                fire_scat(1, m)
            for m in range(K):
                drain_scat(1, m)
            return carry

        lax.fori_loop(0, nj, step, 0)
        plsc.subcore_barrier()

        def write_rows(r0, npt):
            pltpu.sync_copy(degs_sh.at[pl.ds(r0, npt)], zb.at[pl.ds(0, npt)])
            pltpu.sync_copy(zb.at[pl.ds(0, npt)],
                            degs_hbm.at[pl.ds(c * n2 + r0, npt)])

        _tile_node_range(s, n2, write_rows)

    return deg_kernel(idx_flat, ones1, zeros1)


def _sc_segsum(h_pad, src2, dst_flat, zeros, n_pad, d):
    n_chunks, ch = src2.shape
    cpt = n_chunks // NW

    mesh = plsc.VectorSubcoreMesh(core_axis_name="c", subcore_axis_name="s")

    @functools.partial(
        pl.kernel,
        mesh=mesh,
        out_type=jax.ShapeDtypeStruct((NC * n_pad, d), jnp.float32),
        scratch_types=dict(
            isrc=pltpu.VMEM((cpt, ch), jnp.int32),
            ixa=pltpu.VMEM((ch,), jnp.int32),
            ixb=pltpu.VMEM((ch,), jnp.int32),
            rows_a=pltpu.VMEM((ch, d), jnp.float32),
            rows_b=pltpu.VMEM((ch, d), jnp.float32),
            acc_sh=pltpu.VMEM_SHARED((n_pad, d), jnp.float32),
            sem_a=pltpu.SemaphoreType.DMA,
            sem_b=pltpu.SemaphoreType.DMA,
            sxa=pltpu.SemaphoreType.DMA,
            sxb=pltpu.SemaphoreType.DMA,
        ),
    )
    def seg_kernel(h_hbm, src_hbm, dstf_hbm, z_hbm, out_hbm,
                   isrc, ixa, ixb, rows_a, rows_b, acc_sh,
                   sem_a, sem_b, sxa, sxb):
        c = lax.axis_index("c")
        s = lax.axis_index("s")
        wid = c * NS + s
        row0 = wid * cpt
        ebase = wid * cpt * ch
        pltpu.sync_copy(src_hbm.at[pl.ds(row0, cpt)], isrc)

        def load_ix(i, ix, sx):
            pltpu.async_copy(dstf_hbm.at[pl.ds(ebase + i * ch, ch)], ix, sx)

        def wait_ix(i, ix, sx):
            pltpu.make_async_copy(dstf_hbm.at[pl.ds(ebase + i * ch, ch)],
                                  ix, sx).wait()

        load_ix(0, ixa, sxa)
        load_ix(1, ixb, sxb)
        pltpu.async_copy(h_hbm.at[isrc.at[0]], rows_a, sem_a)
        _tile_node_range(s, n_pad, lambda r0, npt: pltpu.sync_copy(
            z_hbm.at[pl.ds(r0, npt)], acc_sh.at[pl.ds(r0, npt)]))
        plsc.subcore_barrier()

        def do_chunk(i, rows, sem, ix, sx):
            wait_ix(i, ix, sx)
            pltpu.make_async_copy(h_hbm.at[isrc.at[i]], rows, sem).wait()
            pltpu.sync_copy(rows, acc_sh.at[ix], add=True)

            @pl.when(i + 2 < cpt)
            def _():
                load_ix(i + 2, ix, sx)

        def step(j, carry):
            i0 = 2 * j
            pltpu.async_copy(h_hbm.at[isrc.at[i0 + 1]], rows_b, sem_b)
            do_chunk(i0, rows_a, sem_a, ixa, sxa)

            @pl.when(j < cpt // 2 - 1)
            def _():
                pltpu.async_copy(h_hbm.at[isrc.at[i0 + 2]], rows_a, sem_a)

            do_chunk(i0 + 1, rows_b, sem_b, ixb, sxb)
            return carry

        lax.fori_loop(0, cpt // 2, step, 0)
        plsc.subcore_barrier()
        _tile_node_range(s, n_pad, lambda r0, npt: pltpu.sync_copy(
            acc_sh.at[pl.ds(r0, npt)],
            out_hbm.at[pl.ds(c * n_pad + r0, npt)]))

    return seg_kernel(h_pad, src2, dst_flat, zeros)


def _norms(degblk):
    ns = lax.rsqrt(jnp.clip(degblk[0] + degblk[1], 1.0, None))
    nd = lax.rsqrt(jnp.clip(degblk[2] + degblk[3], 1.0, None))
    return ns, nd


def _tc_matmul(x, w, n, d, bn):
    def body(x_ref, w_ref, o_ref):
        o_ref[...] = jnp.dot(x_ref[...], w_ref[...],
                             preferred_element_type=jnp.float32)

    return pl.pallas_call(
        body,
        grid=(n // bn,),
        in_specs=[
            pl.BlockSpec((bn, d), lambda i: (i, 0)),
            pl.BlockSpec((d, d), lambda i: (0, 0)),
        ],
        out_specs=pl.BlockSpec((bn, d), lambda i: (i, 0)),
        out_shape=jax.ShapeDtypeStruct((n, d), jnp.float32),
    )(x, w)


def _tc_prescale(y, w, degs_tc, n, d, bn, n_out):
    def body(deg_ref, x_ref, w_ref, o_ref):
        ns, _ = _norms(deg_ref[0])
        o_ref[...] = jnp.dot(x_ref[...], w_ref[...],
                             preferred_element_type=jnp.float32) * ns[:, None]

    return pl.pallas_call(
        body,
        grid=(n // bn,),
        in_specs=[
            pl.BlockSpec((1, 4, bn), lambda i: (i, 0, 0)),
            pl.BlockSpec((bn, d), lambda i: (i, 0)),
            pl.BlockSpec((d, d), lambda i: (0, 0)),
        ],
        out_specs=pl.BlockSpec((bn, d), lambda i: (i, 0)),
        out_shape=jax.ShapeDtypeStruct((n_out, d), jnp.float32),
    )(degs_tc, y, w)


def _tc_mid(part, degs_tc, w2, b1, n, d, bn, n_out):
    def body(p_ref, deg_ref, w_ref, b_ref, o_ref):
        ns, nd = _norms(deg_ref[0])
        h2 = jnp.maximum((p_ref[0] + p_ref[1]) * nd[:, None] + b_ref[...],
                         0.0)
        o_ref[...] = jnp.dot(
            h2, w_ref[...], preferred_element_type=jnp.float32) * ns[:, None]

    return pl.pallas_call(
        body,
        grid=(n // bn,),
        in_specs=[
            pl.BlockSpec((NC, bn, d), lambda i: (0, i, 0)),
            pl.BlockSpec((1, 4, bn), lambda i: (i, 0, 0)),
            pl.BlockSpec((d, d), lambda i: (0, 0)),
            pl.BlockSpec((1, d), lambda i: (0, 0)),
        ],
        out_specs=pl.BlockSpec((bn, d), lambda i: (i, 0)),
        out_shape=jax.ShapeDtypeStruct((n_out, d), jnp.float32),
    )(part, degs_tc, w2, b1)


def _tc_final(part, degs_tc, b, n, d, bn):
    def body(p_ref, deg_ref, b_ref, o_ref):
        _, nd = _norms(deg_ref[0])
        o_ref[...] = (p_ref[0] + p_ref[1]) * nd[:, None] + b_ref[...]

    return pl.pallas_call(
        body,
        grid=(n // bn,),
        in_specs=[
            pl.BlockSpec((NC, bn, d), lambda i: (0, i, 0)),
            pl.BlockSpec((1, 4, bn), lambda i: (i, 0, 0)),
            pl.BlockSpec((1, d), lambda i: (0, 0)),
        ],
        out_specs=pl.BlockSpec((bn, d), lambda i: (i, 0)),
        out_shape=jax.ShapeDtypeStruct((n, d), jnp.float32),
    )(part, degs_tc, b)


def _pad_edges(idx, e_pad, n):
    pad = e_pad - idx.shape[0]
    pidx = (n + (jnp.arange(pad, dtype=jnp.int32) % NPAD)).astype(idx.dtype)
    return jnp.concatenate([idx, pidx])


def kernel(feat, edge_index, W1, b1, W2, b2):
    n, d = feat.shape
    e = edge_index.shape[1]
    n_pad = n + NPAD

    cpt = -(-e // (NW * CH))
    cpt += cpt % 2
    e_pad = NW * cpt * CH

    src_flat = _pad_edges(edge_index[0], e_pad, n)
    dst_flat = _pad_edges(edge_index[1], e_pad, n)
    src2 = src_flat.reshape(e_pad // CH, CH)
    dst2 = dst_flat.reshape(e_pad // CH, CH)
    idx_flat = jnp.stack([src2, dst2 + n_pad], axis=1).reshape(-1)
    zeros = jnp.zeros((n_pad, d), jnp.float32)
    zeros1 = jnp.zeros((2 * n_pad,), jnp.float32)
    ones1 = jnp.ones((CH,), jnp.float32)
    b1r = b1.reshape(1, d)
    b2r = b2.reshape(1, d)

    bn = 2000
    degs = _sc_degrees(idx_flat, ones1, zeros1, n_pad).reshape(NC, 2, n_pad)
    degs_tc = (degs[:, :, :n].transpose(1, 0, 2).reshape(4, n // bn, bn)
               .transpose(1, 0, 2))

    h1p = _tc_prescale(feat, W1, degs_tc, n, d, bn, n_pad)
    p1 = _sc_segsum(h1p, src2, dst_flat, zeros, n_pad, d)
    p1 = p1.reshape(NC, n_pad, d)
    q2p = _tc_mid(p1, degs_tc, W2, b1r, n, d, bn, n_pad)
    p2 = _sc_segsum(q2p, src2, dst_flat, zeros, n_pad, d)
    p2 = p2.reshape(NC, n_pad, d)
    out = _tc_final(p2, degs_tc, b2r, n, d, bn)
    return out
